# baseline (speedup 1.0000x reference)
import functools

import jax
import jax.numpy as jnp
from jax import lax
from jax.experimental import pallas as pl
from jax.experimental.pallas import tpu as pltpu
from jax.experimental.pallas import tpu_sc as plsc

K_POOL = 3
BATCH, SEQ, CH = 4, 8192, 768
NUM_CORES, NUM_SUBCORES, LANES = 2, 16, 16
SC_BATCH = 2
SC_ROWS = 6144
SLICES = NUM_SUBCORES
ROWS_PER_WORKER = SC_ROWS // SLICES
CHUNK = 64
NCHUNK = ROWS_PER_WORKER // CHUNK
GROUPS = CH // LANES
QUAD = 4
BLK = 128
NBLK = CH // BLK
TC_CHUNK = 512
FULL_CHUNKS = SEQ // TC_CHUNK
TAIL_CHUNKS = (SEQ - SC_ROWS) // TC_CHUNK
NEG_INF = float("-inf")


def _quad_top3(a, b, c, d):
    h1, l1 = jnp.maximum(a, b), jnp.minimum(a, b)
    h2, l2 = jnp.maximum(c, d), jnp.minimum(c, d)
    q1, hl = jnp.maximum(h1, h2), jnp.minimum(h1, h2)
    ml = jnp.maximum(l1, l2)
    return q1, jnp.maximum(hl, ml), jnp.minimum(hl, ml)


def _merge_top3(m1, m2, m3, q1, q2, q3):
    r1 = jnp.maximum(m1, q1)
    r2 = jnp.maximum(jnp.minimum(m1, q1), jnp.maximum(m2, q2))
    r3 = jnp.maximum(jnp.maximum(m3, q3),
                     jnp.maximum(jnp.minimum(m1, q2), jnp.minimum(m2, q1)))
    return r1, r2, r3


def _sc_body(x_hbm, out_hbm, part_hbm, buf0, buf1, acc, mbuf, tvec, sem0,
             sem1):
    cid = lax.axis_index("c")
    sid = lax.axis_index("s")
    b = cid
    r0 = sid * ROWS_PER_WORKER

    def init_acc(g, _):
        neg = jnp.full((LANES,), NEG_INF, jnp.float32)
        for j in range(K_POOL):
            acc[j, pl.ds(g * LANES, LANES)] = neg
        return 0

    lax.fori_loop(0, GROUPS, init_acc, 0)

    def chunk_src(idx):
        return x_hbm.at[b, pl.ds(r0 + idx * CHUNK, CHUNK), :]

    pltpu.async_copy(chunk_src(0), buf0, sem0)
    pltpu.async_copy(chunk_src(1), buf1, sem1)

    def compute(buf):
        for q in range(GROUPS // QUAD):
            cols = [(q * QUAD + j) * LANES for j in range(QUAD)]
            init = []
            for c0 in cols:
                init += [acc[j, pl.ds(c0, LANES)] for j in range(K_POOL)]

            def quad_body(t, carry, cols=cols):
                r = t * 4
                out = []
                for j, c0 in enumerate(cols):
                    vals = [buf[r + i, pl.ds(c0, LANES)] for i in range(4)]
                    q123 = _quad_top3(*vals)
                    out += list(_merge_top3(*carry[3 * j:3 * j + 3], *q123))
                return tuple(out)

            res = lax.fori_loop(0, CHUNK // 4, quad_body, tuple(init))
            for j, c0 in enumerate(cols):
                for k in range(K_POOL):
                    acc[k, pl.ds(c0, LANES)] = res[3 * j + k]

    def outer(i, _):
        for ph, (buf, sem) in enumerate(((buf0, sem0), (buf1, sem1))):
            idx = i * 2 + ph
            pltpu.make_async_copy(chunk_src(0), buf, sem).wait()
            compute(buf)

            @pl.when(i < NCHUNK // 2 - 1)
            def _():
                pltpu.async_copy(chunk_src(idx + 2), buf, sem)

        return 0

    lax.fori_loop(0, NCHUNK // 2, outer, 0)

    pltpu.sync_copy(acc, part_hbm.at[b, sid])
    plsc.subcore_barrier()

    @pl.when(sid < NBLK)
    def _merge():
        c0 = pl.multiple_of(sid * BLK, BLK)
        pltpu.sync_copy(part_hbm.at[b, :, :, pl.ds(c0, BLK)], mbuf)
        for g in range(BLK // LANES):
            cand = [mbuf[r, k, pl.ds(g * LANES, LANES)]
                    for r in range(SLICES) for k in range(K_POOL)]
            m = _quad_top3(*cand[0:4])
            for t in range(4, len(cand), 4):
                m = _merge_top3(*m, *_quad_top3(*cand[t:t + 4]))
            for k in range(K_POOL):
                tvec[k, pl.ds(g * LANES, LANES)] = m[k]
        for k in range(K_POOL):
            pltpu.sync_copy(
                tvec.at[k],
                out_hbm.at[pl.ds((b * K_POOL + k) * CH + c0, BLK)])


def _chunk_top3(x):
    m1 = jnp.max(x, axis=0, keepdims=True)
    c1 = jnp.sum((x == m1).astype(jnp.float32), axis=0, keepdims=True)
    s2 = jnp.max(jnp.where(x < m1, x, NEG_INF), axis=0, keepdims=True)
    c2 = jnp.sum((x == s2).astype(jnp.float32), axis=0, keepdims=True)
    s3 = jnp.max(jnp.where(x < s2, x, NEG_INF), axis=0, keepdims=True)
    m2 = jnp.where(c1 >= 2, m1, s2)
    m3 = jnp.where(c1 >= 3, m1,
                   jnp.where((c1 == 2) | (c2 >= 2), s2, s3))
    return m1, m2, m3


def _tc_body(x_ref, o_ref):
    i = pl.program_id(0)
    first = ((i == 0) | (i == FULL_CHUNKS) | (i == 2 * FULL_CHUNKS)
             | (i == 2 * FULL_CHUNKS + TAIL_CHUNKS))
    m1, m2, m3 = _chunk_top3(x_ref[0])

    @pl.when(first)
    def _():
        o_ref[0] = jnp.concatenate([m1, m2, m3], axis=0)

    @pl.when(jnp.logical_not(first))
    def _():
        prev = o_ref[0]
        r = _merge_top3(prev[0:1], prev[1:2], prev[2:3], m1, m2, m3)
        o_ref[0] = jnp.concatenate(r, axis=0)


def _tc_merge_body(a_ref, b_ref, o_ref):
    a = a_ref[0]
    b = b_ref[0]
    r = _merge_top3(a[0:1], a[1:2], a[2:3], b[0:1], b[1:2], b[2:3])
    o_ref[0] = jnp.concatenate(r, axis=0)


@jax.jit
def kernel(inputs):
    mesh = plsc.VectorSubcoreMesh(core_axis_name="c", subcore_axis_name="s")
    sc_run = functools.partial(
        pl.kernel,
        out_type=(
            jax.ShapeDtypeStruct((SC_BATCH * K_POOL * CH,), jnp.float32),
            jax.ShapeDtypeStruct((SC_BATCH, SLICES, K_POOL, CH), jnp.float32),
        ),
        mesh=mesh,
        scratch_types=[
            pltpu.VMEM((CHUNK, CH), jnp.float32),
            pltpu.VMEM((CHUNK, CH), jnp.float32),
            pltpu.VMEM((K_POOL, CH), jnp.float32),
            pltpu.VMEM((SLICES, K_POOL, BLK), jnp.float32),
            pltpu.VMEM((K_POOL, BLK), jnp.float32),
            pltpu.SemaphoreType.DMA,
            pltpu.SemaphoreType.DMA,
        ],
    )(_sc_body)
    sc_out, _ = sc_run(inputs)
    sc_out = sc_out.reshape(SC_BATCH, K_POOL, CH)

    def x_map(i):
        in_b23 = i < 2 * FULL_CHUNKS
        b = jnp.where(in_b23, SC_BATCH + i // FULL_CHUNKS,
                      (i - 2 * FULL_CHUNKS) // TAIL_CHUNKS)
        s = jnp.where(in_b23, i % FULL_CHUNKS,
                      SC_ROWS // TC_CHUNK + (i - 2 * FULL_CHUNKS) % TAIL_CHUNKS)
        return (b, s, 0)

    def o_map(i):
        in_b23 = i < 2 * FULL_CHUNKS
        return (jnp.where(in_b23, i // FULL_CHUNKS,
                          2 + (i - 2 * FULL_CHUNKS) // TAIL_CHUNKS), 0, 0)

    tc_out = pl.pallas_call(
        _tc_body,
        grid=(2 * FULL_CHUNKS + 2 * TAIL_CHUNKS,),
        in_specs=[pl.BlockSpec((1, TC_CHUNK, CH), x_map)],
        out_specs=pl.BlockSpec((1, K_POOL, CH), o_map),
        out_shape=jax.ShapeDtypeStruct((BATCH, K_POOL, CH), jnp.float32),
    )(inputs)

    merged01 = pl.pallas_call(
        _tc_merge_body,
        grid=(SC_BATCH,),
        in_specs=[pl.BlockSpec((1, K_POOL, CH), lambda i: (i, 0, 0)),
                  pl.BlockSpec((1, K_POOL, CH), lambda i: (i + 2, 0, 0))],
        out_specs=pl.BlockSpec((1, K_POOL, CH), lambda i: (i, 0, 0)),
        out_shape=jax.ShapeDtypeStruct((SC_BATCH, K_POOL, CH), jnp.float32),
    )(sc_out, tc_out)

    full = jnp.concatenate([merged01, tc_out[:2]], axis=0)
    return full.transpose(0, 2, 1).reshape(BATCH, CH * K_POOL)


# speedup vs baseline: 1.0578x; 1.0578x over previous
import functools

import jax
import jax.numpy as jnp
from jax import lax
from jax.experimental import pallas as pl
from jax.experimental.pallas import tpu as pltpu
from jax.experimental.pallas import tpu_sc as plsc

K_POOL = 3
BATCH, SEQ, CH = 4, 8192, 768
NUM_CORES, NUM_SUBCORES, LANES = 2, 16, 16
SC_BATCH = 2
SLICES = NUM_SUBCORES
ROWS_PER_WORKER = SEQ // SLICES
CHUNK = 32
NCHUNK = ROWS_PER_WORKER // CHUNK
GROUPS = CH // LANES
QUAD = 4
BLK = 128
NBLK = CH // BLK
TC_CHUNK = 512
NEG_INF = float("-inf")


def _quad_top3(a, b, c, d):
    h1, l1 = jnp.maximum(a, b), jnp.minimum(a, b)
    h2, l2 = jnp.maximum(c, d), jnp.minimum(c, d)
    q1, hl = jnp.maximum(h1, h2), jnp.minimum(h1, h2)
    ml = jnp.maximum(l1, l2)
    return q1, jnp.maximum(hl, ml), jnp.minimum(hl, ml)


def _merge_top3(m1, m2, m3, q1, q2, q3):
    r1 = jnp.maximum(m1, q1)
    r2 = jnp.maximum(jnp.minimum(m1, q1), jnp.maximum(m2, q2))
    r3 = jnp.maximum(jnp.maximum(m3, q3),
                     jnp.maximum(jnp.minimum(m1, q2), jnp.minimum(m2, q1)))
    return r1, r2, r3


def _sc_body(x_hbm, out_hbm, part_hbm, buf0, buf1, acc, mbuf, tvec, sem0,
             sem1):
    cid = lax.axis_index("c")
    sid = lax.axis_index("s")
    b = cid
    r0 = sid * ROWS_PER_WORKER

    def init_acc(g, _):
        neg = jnp.full((LANES,), NEG_INF, jnp.float32)
        for j in range(K_POOL):
            acc[j, pl.ds(g * LANES, LANES)] = neg
        return 0

    lax.fori_loop(0, GROUPS, init_acc, 0)

    def chunk_src(idx):
        return x_hbm.at[b, pl.ds(r0 + idx * CHUNK, CHUNK), :]

    pltpu.async_copy(chunk_src(0), buf0, sem0)
    pltpu.async_copy(chunk_src(1), buf1, sem1)

    def compute(buf):
        for q in range(GROUPS // QUAD):
            cols = [(q * QUAD + j) * LANES for j in range(QUAD)]
            init = []
            for c0 in cols:
                init += [acc[j, pl.ds(c0, LANES)] for j in range(K_POOL)]

            def quad_body(t, carry, cols=cols):
                r = t * 4
                out = []
                for j, c0 in enumerate(cols):
                    vals = [buf[r + i, pl.ds(c0, LANES)] for i in range(4)]
                    q123 = _quad_top3(*vals)
                    out += list(_merge_top3(*carry[3 * j:3 * j + 3], *q123))
                return tuple(out)

            res = lax.fori_loop(0, CHUNK // 4, quad_body, tuple(init))
            for j, c0 in enumerate(cols):
                for k in range(K_POOL):
                    acc[k, pl.ds(c0, LANES)] = res[3 * j + k]

    def outer(i, _):
        for ph, (buf, sem) in enumerate(((buf0, sem0), (buf1, sem1))):
            idx = i * 2 + ph
            pltpu.make_async_copy(chunk_src(0), buf, sem).wait()
            compute(buf)

            @pl.when(i < NCHUNK // 2 - 1)
            def _():
                pltpu.async_copy(chunk_src(idx + 2), buf, sem)

        return 0

    lax.fori_loop(0, NCHUNK // 2, outer, 0)

    pltpu.sync_copy(acc, part_hbm.at[b, sid])
    plsc.subcore_barrier()

    @pl.when(sid < NBLK)
    def _merge():
        c0 = pl.multiple_of(sid * BLK, BLK)
        pltpu.sync_copy(part_hbm.at[b, :, :, pl.ds(c0, BLK)], mbuf)
        for g in range(BLK // LANES):
            cand = [mbuf[r, k, pl.ds(g * LANES, LANES)]
                    for r in range(SLICES) for k in range(K_POOL)]
            m = _quad_top3(*cand[0:4])
            for t in range(4, len(cand), 4):
                m = _merge_top3(*m, *_quad_top3(*cand[t:t + 4]))
            for k in range(K_POOL):
                tvec[k, pl.ds(g * LANES, LANES)] = m[k]
        for k in range(K_POOL):
            pltpu.sync_copy(
                tvec.at[k],
                out_hbm.at[pl.ds((b * K_POOL + k) * CH + c0, BLK)])


def _tc_body(x_ref, o_ref):
    s = pl.program_id(1)
    x = x_ref[0]
    m1 = jnp.max(x, axis=0, keepdims=True)
    c1 = jnp.sum((x == m1).astype(jnp.float32), axis=0, keepdims=True)
    s2 = jnp.max(jnp.where(x < m1, x, NEG_INF), axis=0, keepdims=True)
    c2 = jnp.sum((x == s2).astype(jnp.float32), axis=0, keepdims=True)
    s3 = jnp.max(jnp.where(x < s2, x, NEG_INF), axis=0, keepdims=True)
    m2 = jnp.where(c1 >= 2, m1, s2)
    m3 = jnp.where(c1 >= 3, m1,
                   jnp.where((c1 == 2) | (c2 >= 2), s2, s3))

    @pl.when(s == 0)
    def _():
        o_ref[0] = jnp.concatenate([m1, m2, m3], axis=0)

    @pl.when(s > 0)
    def _():
        prev = o_ref[0]
        r = _merge_top3(prev[0:1], prev[1:2], prev[2:3], m1, m2, m3)
        o_ref[0] = jnp.concatenate(r, axis=0)


@jax.jit
def kernel(inputs):
    mesh = plsc.VectorSubcoreMesh(core_axis_name="c", subcore_axis_name="s")
    sc_run = functools.partial(
        pl.kernel,
        out_type=(
            jax.ShapeDtypeStruct((SC_BATCH * K_POOL * CH,), jnp.float32),
            jax.ShapeDtypeStruct((SC_BATCH, SLICES, K_POOL, CH), jnp.float32),
        ),
        mesh=mesh,
        scratch_types=[
            pltpu.VMEM((CHUNK, CH), jnp.float32),
            pltpu.VMEM((CHUNK, CH), jnp.float32),
            pltpu.VMEM((K_POOL, CH), jnp.float32),
            pltpu.VMEM((SLICES, K_POOL, BLK), jnp.float32),
            pltpu.VMEM((K_POOL, BLK), jnp.float32),
            pltpu.SemaphoreType.DMA,
            pltpu.SemaphoreType.DMA,
        ],
    )(_sc_body)
    sc_out, _ = sc_run(inputs)
    sc_out = sc_out.reshape(SC_BATCH, K_POOL, CH)

    tc_out = pl.pallas_call(
        _tc_body,
        grid=(BATCH - SC_BATCH, SEQ // TC_CHUNK),
        in_specs=[pl.BlockSpec((1, TC_CHUNK, CH),
                               lambda b, s: (b + SC_BATCH, s, 0))],
        out_specs=pl.BlockSpec((1, K_POOL, CH), lambda b, s: (b, 0, 0)),
        out_shape=jax.ShapeDtypeStruct((BATCH - SC_BATCH, K_POOL, CH),
                                       jnp.float32),
    )(inputs)

    full = jnp.concatenate([sc_out, tc_out], axis=0)
    return full.transpose(0, 2, 1).reshape(BATCH, CH * K_POOL)


# speedup vs baseline: 1.1563x; 1.0931x over previous
import functools

import jax
import jax.numpy as jnp
from jax import lax
from jax.experimental import pallas as pl
from jax.experimental.pallas import tpu as pltpu
from jax.experimental.pallas import tpu_sc as plsc

K_POOL = 3
BATCH, SEQ, CH = 4, 8192, 768
NUM_CORES, NUM_SUBCORES, LANES = 2, 16, 16
SC_BATCH = 2
SLICES = NUM_SUBCORES
ROWS_PER_WORKER = SEQ // SLICES
CHUNK = 64
NCHUNK = ROWS_PER_WORKER // CHUNK
GROUPS = CH // LANES
QUAD = 4
BLK = 128
NBLK = CH // BLK
TC_CHUNK = 512
NEG_INF = float("-inf")


def _quad_top3(a, b, c, d):
    h1, l1 = jnp.maximum(a, b), jnp.minimum(a, b)
    h2, l2 = jnp.maximum(c, d), jnp.minimum(c, d)
    q1, hl = jnp.maximum(h1, h2), jnp.minimum(h1, h2)
    ml = jnp.maximum(l1, l2)
    return q1, jnp.maximum(hl, ml), jnp.minimum(hl, ml)


def _merge_top3(m1, m2, m3, q1, q2, q3):
    r1 = jnp.maximum(m1, q1)
    r2 = jnp.maximum(jnp.minimum(m1, q1), jnp.maximum(m2, q2))
    r3 = jnp.maximum(jnp.maximum(m3, q3),
                     jnp.maximum(jnp.minimum(m1, q2), jnp.minimum(m2, q1)))
    return r1, r2, r3


def _sc_body(x_hbm, out_hbm, part_hbm, buf0, buf1, acc, mbuf, tvec, sem0,
             sem1):
    cid = lax.axis_index("c")
    sid = lax.axis_index("s")
    b = cid
    r0 = sid * ROWS_PER_WORKER

    def chunk_src(idx):
        return x_hbm.at[b, pl.ds(r0 + idx * CHUNK, CHUNK), :]

    pltpu.async_copy(chunk_src(0), buf0, sem0)
    pltpu.async_copy(chunk_src(1), buf1, sem1)

    def init_acc(g, _):
        neg = jnp.full((LANES,), NEG_INF, jnp.float32)
        for j in range(K_POOL):
            acc[j, pl.ds(g * LANES, LANES)] = neg
        return 0

    lax.fori_loop(0, GROUPS, init_acc, 0)

    def compute(buf):
        for q in range(GROUPS // QUAD):
            cols = [(q * QUAD + j) * LANES for j in range(QUAD)]
            init = []
            for c0 in cols:
                init += [acc[j, pl.ds(c0, LANES)] for j in range(K_POOL)]

            def quad_body(t, carry, cols=cols):
                r = t * 4
                out = []
                for j, c0 in enumerate(cols):
                    vals = [buf[r + i, pl.ds(c0, LANES)] for i in range(4)]
                    q123 = _quad_top3(*vals)
                    out += list(_merge_top3(*carry[3 * j:3 * j + 3], *q123))
                return tuple(out)

            res = lax.fori_loop(0, CHUNK // 4, quad_body, tuple(init))
            for j, c0 in enumerate(cols):
                for k in range(K_POOL):
                    acc[k, pl.ds(c0, LANES)] = res[3 * j + k]

    def outer(i, _):
        for ph, (buf, sem) in enumerate(((buf0, sem0), (buf1, sem1))):
            idx = i * 2 + ph
            pltpu.make_async_copy(chunk_src(0), buf, sem).wait()
            compute(buf)

            @pl.when(i < NCHUNK // 2 - 1)
            def _():
                pltpu.async_copy(chunk_src(idx + 2), buf, sem)

        return 0

    lax.fori_loop(0, NCHUNK // 2, outer, 0)

    pltpu.sync_copy(acc, part_hbm.at[b, sid])
    plsc.subcore_barrier()

    @pl.when(sid < NBLK)
    def _merge():
        c0 = pl.multiple_of(sid * BLK, BLK)
        pltpu.sync_copy(part_hbm.at[b, :, :, pl.ds(c0, BLK)], mbuf)
        for g in range(BLK // LANES):
            cand = [mbuf[r, k, pl.ds(g * LANES, LANES)]
                    for r in range(SLICES) for k in range(K_POOL)]
            m = _quad_top3(*cand[0:4])
            for t in range(4, len(cand), 4):
                m = _merge_top3(*m, *_quad_top3(*cand[t:t + 4]))
            for k in range(K_POOL):
                tvec[k, pl.ds(g * LANES, LANES)] = m[k]
        for k in range(K_POOL):
            pltpu.sync_copy(
                tvec.at[k],
                out_hbm.at[pl.ds((b * K_POOL + k) * CH + c0, BLK)])


def _tc_body(x_ref, o_ref):
    s = pl.program_id(1)
    x = x_ref[0]
    m1 = jnp.max(x, axis=0, keepdims=True)
    c1 = jnp.sum((x == m1).astype(jnp.float32), axis=0, keepdims=True)
    s2 = jnp.max(jnp.where(x < m1, x, NEG_INF), axis=0, keepdims=True)
    c2 = jnp.sum((x == s2).astype(jnp.float32), axis=0, keepdims=True)
    s3 = jnp.max(jnp.where(x < s2, x, NEG_INF), axis=0, keepdims=True)
    m2 = jnp.where(c1 >= 2, m1, s2)
    m3 = jnp.where(c1 >= 3, m1,
                   jnp.where((c1 == 2) | (c2 >= 2), s2, s3))

    @pl.when(s == 0)
    def _():
        o_ref[0] = jnp.concatenate([m1, m2, m3], axis=0)

    @pl.when(s > 0)
    def _():
        prev = o_ref[0]
        r = _merge_top3(prev[0:1], prev[1:2], prev[2:3], m1, m2, m3)
        o_ref[0] = jnp.concatenate(r, axis=0)


@jax.jit
def kernel(inputs):
    mesh = plsc.VectorSubcoreMesh(core_axis_name="c", subcore_axis_name="s")
    sc_run = functools.partial(
        pl.kernel,
        out_type=(
            jax.ShapeDtypeStruct((SC_BATCH * K_POOL * CH,), jnp.float32),
            jax.ShapeDtypeStruct((SC_BATCH, SLICES, K_POOL, CH), jnp.float32),
        ),
        mesh=mesh,
        scratch_types=[
            pltpu.VMEM((CHUNK, CH), jnp.float32),
            pltpu.VMEM((CHUNK, CH), jnp.float32),
            pltpu.VMEM((K_POOL, CH), jnp.float32),
            pltpu.VMEM((SLICES, K_POOL, BLK), jnp.float32),
            pltpu.VMEM((K_POOL, BLK), jnp.float32),
            pltpu.SemaphoreType.DMA,
            pltpu.SemaphoreType.DMA,
        ],
    )(_sc_body)
    sc_out, _ = sc_run(inputs)
    sc_out = sc_out.reshape(SC_BATCH, K_POOL, CH)

    tc_out = pl.pallas_call(
        _tc_body,
        grid=(BATCH - SC_BATCH, SEQ // TC_CHUNK),
        in_specs=[pl.BlockSpec((1, TC_CHUNK, CH),
                               lambda b, s: (b + SC_BATCH, s, 0))],
        out_specs=pl.BlockSpec((1, K_POOL, CH), lambda b, s: (b, 0, 0)),
        out_shape=jax.ShapeDtypeStruct((BATCH - SC_BATCH, K_POOL, CH),
                                       jnp.float32),
    )(inputs)

    full = jnp.concatenate([sc_out, tc_out], axis=0)
    return full.transpose(0, 2, 1).reshape(BATCH, CH * K_POOL)
